# baseline (speedup 1.0000x reference)
import functools

import jax
import jax.numpy as jnp
from jax import lax
from jax.experimental import pallas as pl
from jax.experimental.pallas import tpu as pltpu
from jax.experimental.pallas import tpu_sc as plsc

N = 10000
D = 128
E = 320000
NC = 2
NS = 16
L = 16
NW = NC * NS
EW = E // NW
CB = 400
NCHUNK = EW // CB
NG = CB // L
NPT = 624
NRC = 104
NRCH = NPT // NRC


def _make_decoder():
    mesh = plsc.VectorSubcoreMesh(core_axis_name="c", subcore_axis_name="s")

    @functools.partial(
        pl.kernel,
        mesh=mesh,
        compiler_params=pltpu.CompilerParams(needs_layout_passes=False),
        out_type=jax.ShapeDtypeStruct((E,), jnp.float32),
        scratch_types=[
            pltpu.VMEM((CB,), jnp.int32),
            pltpu.VMEM((CB,), jnp.int32),
            pltpu.VMEM((CB,), jnp.int32),
            pltpu.VMEM((CB,), jnp.int32),
            pltpu.VMEM((CB, D), jnp.float32),
            pltpu.VMEM((CB, D), jnp.float32),
            pltpu.VMEM((CB,), jnp.float32),
            pltpu.VMEM((CB,), jnp.float32),
            pltpu.VMEM((N,), jnp.float32),
            pltpu.VMEM((NPT + L,), jnp.float32),
            pltpu.VMEM_SHARED((N,), jnp.float32),
            pltpu.SemaphoreType.DMA,
            pltpu.SemaphoreType.DMA,
            pltpu.SemaphoreType.DMA,
            pltpu.SemaphoreType.DMA,
            pltpu.SemaphoreType.DMA,
            pltpu.SemaphoreType.DMA,
        ],
    )
    def decoder(x_hbm, src_hbm, dst_hbm, out_hbm,
                isa, ida, isb, idb, ba, bb, oba, obb, norms_v, nloc,
                norms_sh, ssa, saa, ssb, sab, soa, sob):
        cid = lax.axis_index("c")
        sid = lax.axis_index("s")
        wid = sid * NC + cid
        base = wid * EW
        last_lane = lax.iota(jnp.int32, L) == (L - 1)

        def norm_rows(nrows, src_row0, dst_loc0):
            pltpu.sync_copy(x_hbm.at[pl.ds(src_row0, nrows)],
                            ba.at[pl.ds(0, nrows)])

            def nrow_body(r, carry2):
                u = ba[r, pl.ds(0, L)]
                acc = u * u
                for c in range(1, D // L):
                    u = ba[r, pl.ds(c * L, L)]
                    acc = acc + u * u
                tot = plsc.cumsum(acc)
                ridx = jnp.full((L,), 0, jnp.int32) + (dst_loc0 + r)
                plsc.store_scatter(nloc, [ridx], tot, mask=last_lane)
                return carry2

            lax.fori_loop(0, nrows, nrow_body, 0)

        def nchunk_body(k, carry):
            norm_rows(NRC, sid * NPT + k * NRC, k * NRC)
            return carry

        lax.fori_loop(0, NRCH, nchunk_body, 0)

        @pl.when(sid == NS - 1)
        def _():
            norm_rows(L, N - L, NPT)

        pltpu.sync_copy(nloc.at[pl.ds(0, NPT)],
                        norms_sh.at[pl.ds(sid * NPT, NPT)])

        @pl.when(sid == NS - 1)
        def _():
            pltpu.sync_copy(nloc.at[pl.ds(NPT, L)],
                            norms_sh.at[pl.ds(N - L, L)])

        plsc.subcore_barrier()
        pltpu.sync_copy(norms_sh, norms_v)

        def copy_idx(i, isv, idv):
            off = base + i * CB
            pltpu.sync_copy(src_hbm.at[pl.ds(off, CB)], isv)
            pltpu.sync_copy(dst_hbm.at[pl.ds(off, CB)], idv)

        def start_src(b, isv, ss):
            pltpu.async_copy(x_hbm.at[isv], b, ss)

        def start_add(b, idv, sa):
            pltpu.async_copy(x_hbm.at[idv], b, sa, add=True)

        def wait_rows(b, s):
            pltpu.make_async_copy(x_hbm.at[pl.ds(0, CB)], b, s).wait()

        def wait_out(ob, so):
            pltpu.make_async_copy(
                ob, out_hbm.at[pl.ds(base, CB)], so).wait()

        def compute(i, b, isv, idv, ob, so):
            def edge_body(q, carry):
                for uu in range(4):
                    e = q * 4 + uu
                    u = b[e, pl.ds(0, L)]
                    acc = u * u
                    for c in range(1, D // L):
                        u = b[e, pl.ds(c * L, L)]
                        acc = acc + u * u
                    tot = plsc.cumsum(acc)
                    eidx = jnp.full((L,), 0, jnp.int32) + e
                    plsc.store_scatter(ob, [eidx], tot, mask=last_lane)
                return carry

            lax.fori_loop(0, CB // 4, edge_body, 0)

            def sig_body(k, carry):
                dot2 = ob[pl.ds(k * L, L)]
                si = isv[pl.ds(k * L, L)]
                ti = idv[pl.ds(k * L, L)]
                ns = plsc.load_gather(norms_v, [si])
                nt = plsc.load_gather(norms_v, [ti])
                v = 0.5 * (dot2 - ns - nt)
                ob[pl.ds(k * L, L)] = 1.0 / (1.0 + jnp.exp(-v)) + 1e-15
                return carry

            lax.fori_loop(0, NG, sig_body, 0)
            pltpu.async_copy(ob, out_hbm.at[pl.ds(base + i * CB, CB)], so)

        copy_idx(0, isa, ida)
        start_src(ba, isa, ssa)
        copy_idx(1, isb, idb)
        start_src(bb, isb, ssb)
        wait_rows(ba, ssa)
        start_add(ba, ida, saa)

        def pair_body(j, carry):
            i0 = 2 * j
            wait_rows(bb, ssb)
            start_add(bb, idb, sab)
            wait_rows(ba, saa)

            @pl.when(j > 0)
            def _():
                wait_out(oba, soa)

            compute(i0, ba, isa, ida, oba, soa)

            @pl.when(i0 + 2 < NCHUNK)
            def _():
                copy_idx(i0 + 2, isa, ida)
                start_src(ba, isa, ssa)

            wait_rows(bb, sab)

            @pl.when(j > 0)
            def _():
                wait_out(obb, sob)

            compute(i0 + 1, bb, isb, idb, obb, sob)

            @pl.when(i0 + 3 < NCHUNK)
            def _():
                copy_idx(i0 + 3, isb, idb)
                start_src(bb, isb, ssb)

            @pl.when(i0 + 2 < NCHUNK)
            def _():
                wait_rows(ba, ssa)
                start_add(ba, ida, saa)

            return carry

        lax.fori_loop(0, NCHUNK // 2, pair_body, 0)
        wait_rows(ba, saa)
        wait_out(oba, soa)
        compute(NCHUNK - 1, ba, isa, ida, oba, soa)
        wait_out(oba, soa)
        wait_out(obb, sob)

    return decoder


_decoder = _make_decoder()


@jax.jit
def kernel(x, edge_index):
    ei32 = edge_index.astype(jnp.int32)
    adj_pred = _decoder(x, ei32[0], ei32[1])
    return (adj_pred, edge_index)


# speedup vs baseline: 1.4307x; 1.4307x over previous
import functools

import jax
import jax.numpy as jnp
from jax import lax
from jax.experimental import pallas as pl
from jax.experimental.pallas import tpu as pltpu
from jax.experimental.pallas import tpu_sc as plsc

D = 128
E = 320000
NC = 2
NS = 16
L = 16
NW = NC * NS
EW = E // NW
CB = 200
NCHUNK = EW // CB
NG = (CB + L - 1) // L
OB = NG * L


def _make_decoder():
    mesh = plsc.VectorSubcoreMesh(core_axis_name="c", subcore_axis_name="s")

    @functools.partial(
        pl.kernel,
        mesh=mesh,
        compiler_params=pltpu.CompilerParams(needs_layout_passes=False),
        out_type=jax.ShapeDtypeStruct((E,), jnp.float32),
        scratch_types=[
            pltpu.VMEM((EW,), jnp.int32),
            pltpu.VMEM((EW,), jnp.int32),
            pltpu.VMEM((CB, D), jnp.float32),
            pltpu.VMEM((CB, D), jnp.float32),
            pltpu.VMEM((CB, D), jnp.float32),
            pltpu.VMEM((CB, D), jnp.float32),
            pltpu.VMEM((OB,), jnp.float32),
            pltpu.VMEM((OB,), jnp.float32),
            pltpu.SemaphoreType.DMA,
            pltpu.SemaphoreType.DMA,
            pltpu.SemaphoreType.DMA,
            pltpu.SemaphoreType.DMA,
            pltpu.SemaphoreType.DMA,
            pltpu.SemaphoreType.DMA,
        ],
    )
    def decoder(x_hbm, src_hbm, dst_hbm, out_hbm,
                sidx_v, didx_v, sr0, dr0, sr1, dr1, ob0, ob1,
                ss0, sd0, ss1, sd1, so0, so1):
        wid = lax.axis_index("s") * NC + lax.axis_index("c")
        base = wid * EW
        pltpu.sync_copy(src_hbm.at[pl.ds(base, EW)], sidx_v)
        pltpu.sync_copy(dst_hbm.at[pl.ds(base, EW)], didx_v)

        def start(i, sr, dr, ss, sd):
            off = i * CB
            pltpu.async_copy(x_hbm.at[sidx_v.at[pl.ds(off, CB)]], sr, ss)
            pltpu.async_copy(x_hbm.at[didx_v.at[pl.ds(off, CB)]], dr, sd)

        def wait_rows(sr, dr, ss, sd):
            pltpu.make_async_copy(x_hbm.at[pl.ds(0, CB)], sr, ss).wait()
            pltpu.make_async_copy(x_hbm.at[pl.ds(0, CB)], dr, sd).wait()

        def wait_out(ob, so):
            pltpu.make_async_copy(
                ob.at[pl.ds(0, CB)], out_hbm.at[pl.ds(base, CB)], so).wait()

        def compute(i, sr, dr, ob, so):
            off = i * CB

            last_lane = lax.iota(jnp.int32, L) == (L - 1)

            def edge_body(q, carry):
                for u in range(4):
                    e = q * 4 + u
                    acc = sr[e, pl.ds(0, L)] * dr[e, pl.ds(0, L)]
                    for c in range(1, D // L):
                        acc = acc + sr[e, pl.ds(c * L, L)] * dr[e, pl.ds(c * L, L)]
                    tot = plsc.cumsum(acc)
                    eidx = jnp.full((L,), 0, jnp.int32) + e
                    plsc.store_scatter(ob, [eidx], tot, mask=last_lane)
                return carry

            lax.fori_loop(0, CB // 4, edge_body, 0)

            def sig_body(k, carry):
                v = ob[pl.ds(k * L, L)]
                ob[pl.ds(k * L, L)] = 1.0 / (1.0 + jnp.exp(-v)) + 1e-15
                return carry

            lax.fori_loop(0, NG, sig_body, 0)
            pltpu.async_copy(
                ob.at[pl.ds(0, CB)], out_hbm.at[pl.ds(base + off, CB)], so)

        start(0, sr0, dr0, ss0, sd0)
        start(1, sr1, dr1, ss1, sd1)

        def pair_body(j, carry):
            i0 = 2 * j
            wait_rows(sr0, dr0, ss0, sd0)

            @pl.when(j > 0)
            def _():
                wait_out(ob0, so0)

            compute(i0, sr0, dr0, ob0, so0)

            @pl.when(i0 + 2 < NCHUNK)
            def _():
                start(i0 + 2, sr0, dr0, ss0, sd0)

            wait_rows(sr1, dr1, ss1, sd1)

            @pl.when(j > 0)
            def _():
                wait_out(ob1, so1)

            compute(i0 + 1, sr1, dr1, ob1, so1)

            @pl.when(i0 + 3 < NCHUNK)
            def _():
                start(i0 + 3, sr1, dr1, ss1, sd1)

            return carry

        lax.fori_loop(0, NCHUNK // 2, pair_body, 0)
        wait_out(ob0, so0)
        wait_out(ob1, so1)

    return decoder


_decoder = _make_decoder()


@jax.jit
def kernel(x, edge_index):
    ei32 = edge_index.astype(jnp.int32)
    adj_pred = _decoder(x, ei32[0], ei32[1])
    return (adj_pred, edge_index)


# speedup vs baseline: 1.4447x; 1.0098x over previous
import functools

import jax
import jax.numpy as jnp
from jax import lax
from jax.experimental import pallas as pl
from jax.experimental.pallas import tpu as pltpu
from jax.experimental.pallas import tpu_sc as plsc

D = 128
E = 320000
NC = 2
NS = 16
L = 16
NW = NC * NS
EW = E // NW
CB = 200
NCHUNK = EW // CB
NG = (CB + L - 1) // L
OB = NG * L


def _make_decoder():
    mesh = plsc.VectorSubcoreMesh(core_axis_name="c", subcore_axis_name="s")

    @functools.partial(
        pl.kernel,
        mesh=mesh,
        compiler_params=pltpu.CompilerParams(needs_layout_passes=False),
        out_type=jax.ShapeDtypeStruct((E,), jnp.float32),
        scratch_types=[
            pltpu.VMEM((EW,), jnp.int32),
            pltpu.VMEM((EW,), jnp.int32),
            pltpu.VMEM((CB, D), jnp.float32),
            pltpu.VMEM((CB, D), jnp.float32),
            pltpu.VMEM((CB, D), jnp.float32),
            pltpu.VMEM((CB, D), jnp.float32),
            pltpu.VMEM((OB,), jnp.float32),
            pltpu.VMEM((OB,), jnp.float32),
            pltpu.SemaphoreType.DMA,
            pltpu.SemaphoreType.DMA,
            pltpu.SemaphoreType.DMA,
            pltpu.SemaphoreType.DMA,
            pltpu.SemaphoreType.DMA,
            pltpu.SemaphoreType.DMA,
        ],
    )
    def decoder(x_hbm, src_hbm, dst_hbm, out_hbm,
                sidx_v, didx_v, sr0, dr0, sr1, dr1, ob0, ob1,
                ss0, sd0, ss1, sd1, so0, so1):
        wid = lax.axis_index("s") * NC + lax.axis_index("c")
        base = wid * EW
        pltpu.sync_copy(src_hbm.at[pl.ds(base, EW)], sidx_v)
        pltpu.sync_copy(dst_hbm.at[pl.ds(base, EW)], didx_v)

        def start(i, sr, dr, ss, sd):
            off = i * CB
            pltpu.async_copy(x_hbm.at[sidx_v.at[pl.ds(off, CB)]], sr, ss)
            pltpu.async_copy(x_hbm.at[didx_v.at[pl.ds(off, CB)]], dr, sd)

        def wait_rows(sr, dr, ss, sd):
            pltpu.make_async_copy(x_hbm.at[pl.ds(0, CB)], sr, ss).wait()
            pltpu.make_async_copy(x_hbm.at[pl.ds(0, CB)], dr, sd).wait()

        def wait_out(ob, so):
            pltpu.make_async_copy(
                ob.at[pl.ds(0, CB)], out_hbm.at[pl.ds(base, CB)], so).wait()

        def compute(i, sr, dr, ob, so):
            off = i * CB

            last_lane = lax.iota(jnp.int32, L) == (L - 1)

            def edge_body(q, carry):
                for u in range(8):
                    e = q * 8 + u
                    acc = sr[e, pl.ds(0, L)] * dr[e, pl.ds(0, L)]
                    for c in range(1, D // L):
                        acc = acc + sr[e, pl.ds(c * L, L)] * dr[e, pl.ds(c * L, L)]
                    tot = plsc.cumsum(acc)
                    eidx = jnp.full((L,), 0, jnp.int32) + e
                    plsc.store_scatter(ob, [eidx], tot, mask=last_lane)
                return carry

            lax.fori_loop(0, CB // 8, edge_body, 0)

            def sig_body(k, carry):
                v = ob[pl.ds(k * L, L)]
                ob[pl.ds(k * L, L)] = 1.0 / (1.0 + jnp.exp(-v)) + 1e-15
                return carry

            lax.fori_loop(0, NG, sig_body, 0)
            pltpu.async_copy(
                ob.at[pl.ds(0, CB)], out_hbm.at[pl.ds(base + off, CB)], so)

        start(0, sr0, dr0, ss0, sd0)
        start(1, sr1, dr1, ss1, sd1)

        def pair_body(j, carry):
            i0 = 2 * j
            wait_rows(sr0, dr0, ss0, sd0)

            @pl.when(j > 0)
            def _():
                wait_out(ob0, so0)

            compute(i0, sr0, dr0, ob0, so0)

            @pl.when(i0 + 2 < NCHUNK)
            def _():
                start(i0 + 2, sr0, dr0, ss0, sd0)

            wait_rows(sr1, dr1, ss1, sd1)

            @pl.when(j > 0)
            def _():
                wait_out(ob1, so1)

            compute(i0 + 1, sr1, dr1, ob1, so1)

            @pl.when(i0 + 3 < NCHUNK)
            def _():
                start(i0 + 3, sr1, dr1, ss1, sd1)

            return carry

        lax.fori_loop(0, NCHUNK // 2, pair_body, 0)
        wait_out(ob0, so0)
        wait_out(ob1, so1)

    return decoder


_decoder = _make_decoder()


@jax.jit
def kernel(x, edge_index):
    ei32 = edge_index.astype(jnp.int32)
    adj_pred = _decoder(x, ei32[0], ei32[1])
    return (adj_pred, edge_index)
